# baseline (speedup 1.0000x reference)
import functools
import math

import jax
import jax.numpy as jnp
from jax.experimental import pallas as pl
from jax.experimental.pallas import tpu as pltpu

B, S, D = 1, 2048, 768
H = 12
DH = D // H
FF = 512
E = 8
EPS = 1e-5
CQ = 1024


def _attn_body(x_ref, w3_ref, wo_ref, g1_ref, b1_ref, wg_ref,
               h1_ref, gates_ref, z_ref):
    h = pl.program_id(0)

    @pl.when(h < H)
    def _attention_step():
        x = x_ref[...]
        w3 = w3_ref[0]
        q = jnp.dot(x, w3[:, :DH], preferred_element_type=jnp.float32)
        k = jnp.dot(x, w3[:, DH:2 * DH], preferred_element_type=jnp.float32)
        v = jnp.dot(x, w3[:, 2 * DH:], preferred_element_type=jnp.float32)
        wo = wo_ref[0]
        scale = 1.0 / math.sqrt(DH)
        for c in range(S // CQ):
            qc = q[c * CQ:(c + 1) * CQ, :]
            s = jax.lax.dot_general(
                qc, k, (((1,), (1,)), ((), ())),
                preferred_element_type=jnp.float32) * scale
            m = jnp.max(s, axis=1, keepdims=True)
            p = jnp.exp(s - m)
            l = jnp.sum(p, axis=1, keepdims=True)
            o = jnp.dot(p, v, preferred_element_type=jnp.float32) / l
            contrib = jnp.dot(o, wo, preferred_element_type=jnp.float32)
            rows = pl.ds(c * CQ, CQ)

            @pl.when(h == 0)
            def _init():
                z_ref[rows, :] = x[c * CQ:(c + 1) * CQ, :] + contrib

            @pl.when(h > 0)
            def _acc():
                z_ref[rows, :] += contrib

    @pl.when(h == H)
    def _bn_gate_step():
        z = z_ref[...]
        mean = jnp.mean(z, axis=0, keepdims=True)
        zc = z - mean
        var = jnp.mean(zc * zc, axis=0, keepdims=True)
        h1 = zc * jax.lax.rsqrt(var + EPS) * g1_ref[...] + b1_ref[...]
        h1_ref[...] = h1
        logits = jnp.dot(h1, wg_ref[...], preferred_element_type=jnp.float32)
        lanes = jax.lax.broadcasted_iota(jnp.int32, logits.shape, 1)
        logits = jnp.where(lanes < E, logits, -1e30)
        m1 = jnp.max(logits, axis=1, keepdims=True)
        i1 = jnp.min(jnp.where(logits == m1, lanes, 128), axis=1, keepdims=True)
        mask1 = lanes == i1
        l2 = jnp.where(mask1, -1e30, logits)
        m2 = jnp.max(l2, axis=1, keepdims=True)
        i2 = jnp.min(jnp.where(l2 == m2, lanes, 128), axis=1, keepdims=True)
        mask2 = lanes == i2
        e2 = jnp.exp(m2 - m1)
        denom = 1.0 + e2
        g1 = 1.0 / denom
        g2 = e2 / denom
        gates_ref[...] = jnp.where(mask1, g1, 0.0) + jnp.where(mask2, g2, 0.0)


def _moe_body(h1_ref, gates_ref, w1_ref, b1_ref, w2_ref, b2_ref,
              g2_ref, be2_ref, y_ref, z2_ref):
    e = pl.program_id(0)

    @pl.when(e < E)
    def _expert_step():
        hb = h1_ref[...]
        a = jnp.dot(hb.astype(jnp.bfloat16), w1_ref[0],
                    preferred_element_type=jnp.float32)
        a = a + b1_ref[0]
        a = 0.5 * a * (1.0 + jax.lax.erf(a * (1.0 / math.sqrt(2.0))))
        h2 = jnp.dot(a.astype(jnp.bfloat16), w2_ref[0],
                     preferred_element_type=jnp.float32)
        h2 = h2 + b2_ref[0]
        lanes = jax.lax.broadcasted_iota(jnp.int32, gates_ref.shape, 1)
        g = jnp.sum(jnp.where(lanes == e, gates_ref[...], 0.0),
                    axis=1, keepdims=True)
        contrib = g * h2

        @pl.when(e == 0)
        def _init():
            z2_ref[...] = hb + contrib

        @pl.when(e > 0)
        def _acc():
            z2_ref[...] += contrib

    @pl.when(e == E)
    def _bn_step():
        z = z2_ref[...]
        mean = jnp.mean(z, axis=0, keepdims=True)
        zc = z - mean
        var = jnp.mean(zc * zc, axis=0, keepdims=True)
        y_ref[...] = zc * jax.lax.rsqrt(var + EPS) * g2_ref[...] + be2_ref[...]


@jax.jit
def kernel(x, Wq, Wk, Wv, Wo, gamma1, beta1, w_gate, W1e, b1e, W2e, b2e,
           gamma2, beta2):
    xf = x.reshape(S, D)
    w3 = jnp.concatenate([Wq, Wk, Wv], axis=2)
    wg_pad = jnp.pad(w_gate, ((0, 0), (0, 128 - E)))
    g1 = gamma1.reshape(1, D)
    b1 = beta1.reshape(1, D)
    g2 = gamma2.reshape(1, D)
    b2 = beta2.reshape(1, D)
    b1e3 = b1e.reshape(E, 1, FF)
    b2e3 = b2e.reshape(E, 1, D)

    const2 = lambda i: (0, 0)
    head3 = lambda i: (jnp.minimum(i, H - 1), 0, 0)

    h1, gates = pl.pallas_call(
        _attn_body,
        grid=(H + 1,),
        in_specs=[
            pl.BlockSpec((S, D), const2),
            pl.BlockSpec((1, D, 3 * DH), head3),
            pl.BlockSpec((1, DH, D), head3),
            pl.BlockSpec((1, D), const2),
            pl.BlockSpec((1, D), const2),
            pl.BlockSpec((D, 128), const2),
        ],
        out_specs=[
            pl.BlockSpec((S, D), const2),
            pl.BlockSpec((S, 128), const2),
        ],
        out_shape=[
            jax.ShapeDtypeStruct((S, D), jnp.float32),
            jax.ShapeDtypeStruct((S, 128), jnp.float32),
        ],
        scratch_shapes=[pltpu.VMEM((S, D), jnp.float32)],
    )(xf, w3, Wo, g1, b1, wg_pad)

    expert3 = lambda i: (jnp.minimum(i, E - 1), 0, 0)
    y = pl.pallas_call(
        _moe_body,
        grid=(E + 1,),
        in_specs=[
            pl.BlockSpec((S, D), const2),
            pl.BlockSpec((S, 128), const2),
            pl.BlockSpec((1, D, FF), expert3),
            pl.BlockSpec((1, 1, FF), expert3),
            pl.BlockSpec((1, FF, D), expert3),
            pl.BlockSpec((1, 1, D), expert3),
            pl.BlockSpec((1, D), const2),
            pl.BlockSpec((1, D), const2),
        ],
        out_specs=pl.BlockSpec((S, D), const2),
        out_shape=jax.ShapeDtypeStruct((S, D), jnp.float32),
        scratch_shapes=[pltpu.VMEM((S, D), jnp.float32)],
    )(h1, gates, W1e.astype(jnp.bfloat16), b1e3, W2e.astype(jnp.bfloat16),
      b2e3, g2, b2)

    return y.reshape(B, S, D)


# speedup vs baseline: 1.7170x; 1.7170x over previous
import functools
import math

import jax
import jax.numpy as jnp
from jax.experimental import pallas as pl
from jax.experimental.pallas import tpu as pltpu

B, S, D = 1, 2048, 768
H = 12
DH = D // H
FF = 512
E = 8
EPS = 1e-5
CQ = 256
NC = S // CQ


def _attn_body(x_hbm, wq_hbm, wkv_hbm, wo_hbm, g1_ref, b1_ref, wg_ref,
               h1_ref, gates_ref, xs, wq_s, kv_s, wo_s, sem):
    step = pl.program_id(0)

    @pl.when(step == 0)
    def _stage_step():
        pltpu.make_async_copy(x_hbm, xs, sem).start()
        pltpu.make_async_copy(x_hbm, xs, sem).wait()
        pltpu.make_async_copy(wq_hbm, wq_s, sem).start()
        pltpu.make_async_copy(wq_hbm, wq_s, sem).wait()
        pltpu.make_async_copy(wo_hbm, wo_s, sem).start()
        pltpu.make_async_copy(wo_hbm, wo_s, sem).wait()

        def _kv_proj(wkv_vmem):
            pltpu.make_async_copy(wkv_hbm, wkv_vmem, sem).start()
            pltpu.make_async_copy(wkv_hbm, wkv_vmem, sem).wait()
            kv_s[...] = jnp.dot(xs[...], wkv_vmem[...],
                                preferred_element_type=jnp.float32)

        pl.run_scoped(_kv_proj, pltpu.VMEM((D, 2 * D), jnp.float32))

    @pl.when((step >= 1) & (step <= NC))
    def _chunk_step():
        rows = pl.ds((step - 1) * CQ, CQ)
        qc_all = jnp.dot(xs[rows, :], wq_s[...],
                         preferred_element_type=jnp.float32)
        scale = 1.0 / math.sqrt(DH)
        o_cols = []
        for h in range(H):
            qc = qc_all[:, h * DH:(h + 1) * DH]
            k = kv_s[:, h * DH:(h + 1) * DH]
            v = kv_s[:, D + h * DH:D + (h + 1) * DH]
            s = jax.lax.dot_general(
                qc, k, (((1,), (1,)), ((), ())),
                preferred_element_type=jnp.float32) * scale
            m = jnp.max(s, axis=1, keepdims=True)
            p = jnp.exp(s - m)
            l = jnp.sum(p, axis=1, keepdims=True)
            o = jnp.dot(p, v, preferred_element_type=jnp.float32) / l
            o_cols.append(o)
        h1_ref[rows, :] = jnp.concatenate(o_cols, axis=1)

    @pl.when(step == NC + 1)
    def _bn_gate_step():
        z = xs[...] + jnp.dot(h1_ref[...], wo_s[...],
                              preferred_element_type=jnp.float32)
        mean = jnp.mean(z, axis=0, keepdims=True)
        zc = z - mean
        var = jnp.mean(zc * zc, axis=0, keepdims=True)
        h1 = zc * jax.lax.rsqrt(var + EPS) * g1_ref[...] + b1_ref[...]
        h1_ref[...] = h1
        logits = jnp.dot(h1, wg_ref[...], preferred_element_type=jnp.float32)
        lanes = jax.lax.broadcasted_iota(jnp.int32, logits.shape, 1)
        logits = jnp.where(lanes < E, logits, -1e30)
        m1 = jnp.max(logits, axis=1, keepdims=True)
        i1 = jnp.min(jnp.where(logits == m1, lanes, 128), axis=1, keepdims=True)
        mask1 = lanes == i1
        l2 = jnp.where(mask1, -1e30, logits)
        m2 = jnp.max(l2, axis=1, keepdims=True)
        i2 = jnp.min(jnp.where(l2 == m2, lanes, 128), axis=1, keepdims=True)
        mask2 = lanes == i2
        e2 = jnp.exp(m2 - m1)
        denom = 1.0 + e2
        g1 = 1.0 / denom
        g2 = e2 / denom
        gates_ref[...] = jnp.where(mask1, g1, 0.0) + jnp.where(mask2, g2, 0.0)


def _moe_body(h1_ref, gates_ref, w1_ref, b1_ref, w2_ref, b2_ref,
              g2_ref, be2_ref, y_ref, z2_ref):
    e = pl.program_id(0)

    @pl.when(e < E)
    def _expert_step():
        hb = h1_ref[...]
        a = jnp.dot(hb, w1_ref[0], preferred_element_type=jnp.float32)
        a = a + b1_ref[0]
        a = 0.5 * a * (1.0 + jax.lax.erf(a * (1.0 / math.sqrt(2.0))))
        h2 = jnp.dot(a, w2_ref[0], preferred_element_type=jnp.float32)
        h2 = h2 + b2_ref[0]
        lanes = jax.lax.broadcasted_iota(jnp.int32, gates_ref.shape, 1)
        g = jnp.sum(jnp.where(lanes == e, gates_ref[...], 0.0),
                    axis=1, keepdims=True)
        contrib = g * h2

        @pl.when(e == 0)
        def _init():
            z2_ref[...] = hb + contrib

        @pl.when(e > 0)
        def _acc():
            z2_ref[...] += contrib

    @pl.when(e == E)
    def _bn_step():
        z = z2_ref[...]
        mean = jnp.mean(z, axis=0, keepdims=True)
        zc = z - mean
        var = jnp.mean(zc * zc, axis=0, keepdims=True)
        y_ref[...] = zc * jax.lax.rsqrt(var + EPS) * g2_ref[...] + be2_ref[...]


@jax.jit
def kernel(x, Wq, Wk, Wv, Wo, gamma1, beta1, w_gate, W1e, b1e, W2e, b2e,
           gamma2, beta2):
    xf = x.reshape(S, D)
    wqc = Wq.transpose(1, 0, 2).reshape(D, D)
    wkv = jnp.concatenate(
        [Wk.transpose(1, 0, 2).reshape(D, D),
         Wv.transpose(1, 0, 2).reshape(D, D)], axis=1)
    woc = Wo.reshape(D, D)
    wg_pad = jnp.pad(w_gate, ((0, 0), (0, 128 - E)))
    g1 = gamma1.reshape(1, D)
    b1 = beta1.reshape(1, D)
    g2 = gamma2.reshape(1, D)
    b2 = beta2.reshape(1, D)
    b1e3 = b1e.reshape(E, 1, FF)
    b2e3 = b2e.reshape(E, 1, D)

    const2 = lambda i: (0, 0)
    hbm = pl.BlockSpec(memory_space=pl.ANY)

    h1, gates = pl.pallas_call(
        _attn_body,
        grid=(NC + 2,),
        in_specs=[
            hbm, hbm, hbm, hbm,
            pl.BlockSpec((1, D), const2),
            pl.BlockSpec((1, D), const2),
            pl.BlockSpec((D, 128), const2),
        ],
        out_specs=[
            pl.BlockSpec((S, D), const2),
            pl.BlockSpec((S, 128), const2),
        ],
        out_shape=[
            jax.ShapeDtypeStruct((S, D), jnp.float32),
            jax.ShapeDtypeStruct((S, 128), jnp.float32),
        ],
        scratch_shapes=[
            pltpu.VMEM((S, D), jnp.float32),
            pltpu.VMEM((D, D), jnp.float32),
            pltpu.VMEM((S, 2 * D), jnp.float32),
            pltpu.VMEM((D, D), jnp.float32),
            pltpu.SemaphoreType.DMA,
        ],
    )(xf, wqc, wkv, woc, g1, b1, wg_pad)

    expert3 = lambda i: (jnp.minimum(i, E - 1), 0, 0)
    y = pl.pallas_call(
        _moe_body,
        grid=(E + 1,),
        in_specs=[
            pl.BlockSpec((S, D), const2),
            pl.BlockSpec((S, 128), const2),
            pl.BlockSpec((1, D, FF), expert3),
            pl.BlockSpec((1, 1, FF), expert3),
            pl.BlockSpec((1, FF, D), expert3),
            pl.BlockSpec((1, 1, D), expert3),
            pl.BlockSpec((1, D), const2),
            pl.BlockSpec((1, D), const2),
        ],
        out_specs=pl.BlockSpec((S, D), const2),
        out_shape=jax.ShapeDtypeStruct((S, D), jnp.float32),
        scratch_shapes=[pltpu.VMEM((S, D), jnp.float32)],
    )(h1, gates, W1e, b1e3, W2e, b2e3, g2, b2)

    return y.reshape(B, S, D)
